# baseline (speedup 1.0000x reference)
import functools
import math

import jax
import jax.numpy as jnp
from jax.experimental import pallas as pl

ALPHA = 0.1
LAMDA = 0.5


def _cdiv(a, b):
    return (a + b - 1) // b


def _fused_mm(x, w, b, act=None, block_rows=1024):
    n, k = x.shape
    m = w.shape[1]
    b2 = b.reshape(1, m)

    def body(x_ref, w_ref, b_ref, o_ref):
        y = jnp.dot(x_ref[...], w_ref[...], preferred_element_type=jnp.float32)
        y = y + b_ref[...]
        if act == "relu":
            y = jnp.maximum(y, 0.0)
        elif act == "sigmoid":
            y = jax.nn.sigmoid(y)
        o_ref[...] = y

    grid = (_cdiv(n, block_rows),)
    return pl.pallas_call(
        body,
        grid=grid,
        in_specs=[
            pl.BlockSpec((block_rows, k), lambda i: (i, 0)),
            pl.BlockSpec((k, m), lambda i: (0, 0)),
            pl.BlockSpec((1, m), lambda i: (0, 0)),
        ],
        out_specs=pl.BlockSpec((block_rows, m), lambda i: (i, 0)),
        out_shape=jax.ShapeDtypeStruct((n, m), jnp.float32),
    )(x, w, b2)


def _mix_kernel(agg, h0, w, b, beta, block_rows=1024):
    n, m = agg.shape
    b2 = b.reshape(1, m)

    def body(a_ref, h0_ref, w_ref, b_ref, o_ref):
        hi = (1.0 - ALPHA) * a_ref[...] + ALPHA * h0_ref[...]
        y = jnp.dot(hi, w_ref[...], preferred_element_type=jnp.float32) + b_ref[...]
        o_ref[...] = jnp.maximum((1.0 - beta) * hi + beta * y, 0.0)

    grid = (_cdiv(n, block_rows),)
    return pl.pallas_call(
        body,
        grid=grid,
        in_specs=[
            pl.BlockSpec((block_rows, m), lambda i: (i, 0)),
            pl.BlockSpec((block_rows, m), lambda i: (i, 0)),
            pl.BlockSpec((m, m), lambda i: (0, 0)),
            pl.BlockSpec((1, m), lambda i: (0, 0)),
        ],
        out_specs=pl.BlockSpec((block_rows, m), lambda i: (i, 0)),
        out_shape=jax.ShapeDtypeStruct((n, m), jnp.float32),
    )(agg, h0, w, b2)


def _mlp2(x, w1, b1, w2, b2, block_rows=1024):
    n, k = x.shape
    h = w1.shape[1]
    b1r = b1.reshape(1, h)
    b2r = b2.reshape(1, 1)

    def body(x_ref, w1_ref, b1_ref, w2_ref, b2_ref, o_ref):
        y = jnp.dot(x_ref[...], w1_ref[...], preferred_element_type=jnp.float32)
        y = jnp.maximum(y + b1_ref[...], 0.0)
        z = jnp.dot(y, w2_ref[...], preferred_element_type=jnp.float32) + b2_ref[...]
        o_ref[...] = z

    grid = (_cdiv(n, block_rows),)
    out = pl.pallas_call(
        body,
        grid=grid,
        in_specs=[
            pl.BlockSpec((block_rows, k), lambda i: (i, 0)),
            pl.BlockSpec((k, h), lambda i: (0, 0)),
            pl.BlockSpec((1, h), lambda i: (0, 0)),
            pl.BlockSpec((h, 1), lambda i: (0, 0)),
            pl.BlockSpec((1, 1), lambda i: (0, 0)),
        ],
        out_specs=pl.BlockSpec((block_rows, 1), lambda i: (i, 0)),
        out_shape=jax.ShapeDtypeStruct((n, 1), jnp.float32),
    )(x, w1, b1r, w2, b2r)
    return out[:, 0]


def _egnn_block(p, src, dst, h, ef):
    w_in, b_in = p["in"]
    h0 = _fused_mm(h, w_in, b_in, act="relu")
    hcur = h0
    n = h.shape[0]
    for l, lp in enumerate(p["layers"]):
        we, be = lp["edge"]
        wl, bl = lp["lin"]
        gate = _fused_mm(ef, we, be, act="sigmoid", block_rows=8192)
        msg = gate * hcur[src]
        agg = jnp.zeros((n, hcur.shape[1]), jnp.float32).at[dst].add(msg)
        beta = float(math.log(LAMDA / (l + 1) + 1.0))
        hcur = _mix_kernel(agg, h0, wl, bl, beta)
    return hcur


def kernel(edge_index, h, x, adj, efeats, params):
    h = jnp.squeeze(h).astype(jnp.float32)
    x = jnp.squeeze(x)
    src, dst = edge_index[0], edge_index[1]
    dist = jnp.sum((x[src] - x[dst]) ** 2, axis=-1, keepdims=True)
    ef = jnp.concatenate([efeats, dist], axis=-1)

    f1 = _egnn_block(params["b1"], src, dst, h, ef)
    f2 = _egnn_block(params["b2"], src, dst, f1, ef)
    f3 = _egnn_block(params["b3"], src, dst, f2, ef)

    wm, bm = params["proj_middle"]
    middle_proj = _fused_mm(f2, wm, bm)
    ws, bs = params["proj_shallow"]
    shallow_proj = _fused_mm(f1, ws, bs)

    logit1 = _mlp2(f1, *params["cls1"][0], *params["cls1"][1])
    logit2 = _mlp2(f2, *params["cls2"][0], *params["cls2"][1])
    logit3 = _mlp2(f3, *params["cls3"][0], *params["cls3"][1])
    return ((logit3, logit2, logit1), (f3, middle_proj, shallow_proj))


# speedup vs baseline: 1.9356x; 1.9356x over previous
import functools
import math

import jax
import jax.numpy as jnp
from jax import lax
from jax.experimental import pallas as pl
from jax.experimental.pallas import tpu as pltpu
from jax.experimental.pallas import tpu_sc as plsc

ALPHA = 0.1
LAMDA = 0.5

N_NODES = 10000
N_EDGES = 320000
N_TILES = 16
ROWS_PER_TILE = 624
TAIL_ROWS = N_NODES - ROWS_PER_TILE * N_TILES
CHUNK = 80
ZROWS = 104


def _cdiv(a, b):
    return (a + b - 1) // b


def _fused_mm(x, w, b, act=None, block_rows=1024):
    n, k = x.shape
    m = w.shape[1]
    b2 = b.reshape(1, m)

    def body(x_ref, w_ref, b_ref, o_ref):
        y = jnp.dot(x_ref[...], w_ref[...], preferred_element_type=jnp.float32)
        y = y + b_ref[...]
        if act == "relu":
            y = jnp.maximum(y, 0.0)
        o_ref[...] = y

    return pl.pallas_call(
        body,
        grid=(_cdiv(n, block_rows),),
        in_specs=[
            pl.BlockSpec((block_rows, k), lambda i: (i, 0)),
            pl.BlockSpec((k, m), lambda i: (0, 0)),
            pl.BlockSpec((1, m), lambda i: (0, 0)),
        ],
        out_specs=pl.BlockSpec((block_rows, m), lambda i: (i, 0)),
        out_shape=jax.ShapeDtypeStruct((n, m), jnp.float32),
    )(x, w, b2)


def _gate_kernel(ef, we, be, hp, block_rows=8000):
    e, k = ef.shape
    nb = e // block_rows
    be2 = be.reshape(1, we.shape[1])

    def body(ef_ref, we_ref, be_ref, o_ref):
        y = jnp.dot(ef_ref[...], we_ref[...], preferred_element_type=jnp.float32)
        o_ref[...] = jax.nn.sigmoid(y + be_ref[...])

    return pl.pallas_call(
        body,
        grid=(hp, nb),
        in_specs=[
            pl.BlockSpec((block_rows, k), lambda p, i: (i, 0)),
            pl.BlockSpec((k, 128), lambda p, i: (0, p)),
            pl.BlockSpec((1, 128), lambda p, i: (0, p)),
        ],
        out_specs=pl.BlockSpec((block_rows, 128), lambda p, i: (p * nb + i, 0)),
        out_shape=jax.ShapeDtypeStruct((hp * e, 128), jnp.float32),
    )(ef, we, be2)


def _hi_kernel(aggf, h0, hsize, block_rows=1000):
    n, h = h0.shape
    nb = n // block_rows
    hp = hsize // 128

    if hp >= 2:
        def body(a_ref, h0_ref, o_ref):
            o_ref[...] = (1.0 - ALPHA) * a_ref[...] + ALPHA * h0_ref[...]

        return pl.pallas_call(
            body,
            grid=(nb, hp),
            in_specs=[
                pl.BlockSpec((block_rows, 128), lambda i, p: (p * nb + i, 0)),
                pl.BlockSpec((block_rows, 128), lambda i, p: (i, p)),
            ],
            out_specs=pl.BlockSpec((block_rows, 128), lambda i, p: (i, p)),
            out_shape=jax.ShapeDtypeStruct((n, h), jnp.float32),
        )(aggf, h0)

    def body2(a0_ref, a1_ref, h0_ref, o_ref):
        agg = a0_ref[...] + a1_ref[...]
        o_ref[...] = (1.0 - ALPHA) * agg + ALPHA * h0_ref[...]

    return pl.pallas_call(
        body2,
        grid=(nb,),
        in_specs=[
            pl.BlockSpec((block_rows, 128), lambda i: (i, 0)),
            pl.BlockSpec((block_rows, 128), lambda i: (nb + i, 0)),
            pl.BlockSpec((block_rows, 128), lambda i: (i, 0)),
        ],
        out_specs=pl.BlockSpec((block_rows, 128), lambda i: (i, 0)),
        out_shape=jax.ShapeDtypeStruct((n, h), jnp.float32),
    )(aggf, aggf, h0)


def _lin_kernel(hi, wl, bl, beta, block_rows=1000):
    n, h = hi.shape
    b2 = bl.reshape(1, h)

    def body(hi_ref, w_ref, b_ref, o_ref):
        hi_v = hi_ref[...]
        y = jnp.dot(hi_v, w_ref[...], preferred_element_type=jnp.float32) + b_ref[...]
        o_ref[...] = jnp.maximum((1.0 - beta) * hi_v + beta * y, 0.0)

    return pl.pallas_call(
        body,
        grid=(_cdiv(n, block_rows),),
        in_specs=[
            pl.BlockSpec((block_rows, h), lambda i: (i, 0)),
            pl.BlockSpec((h, h), lambda i: (0, 0)),
            pl.BlockSpec((1, h), lambda i: (0, 0)),
        ],
        out_specs=pl.BlockSpec((block_rows, h), lambda i: (i, 0)),
        out_shape=jax.ShapeDtypeStruct((n, h), jnp.float32),
    )(hi, wl, b2)


def _mlp2(x, w1, b1, w2, b2, block_rows=1000):
    n, k = x.shape
    h = w1.shape[1]
    b1r = b1.reshape(1, h)
    b2r = b2.reshape(1, 1)

    def body(x_ref, w1_ref, b1_ref, w2_ref, b2_ref, o_ref):
        y = jnp.dot(x_ref[...], w1_ref[...], preferred_element_type=jnp.float32)
        y = jnp.maximum(y + b1_ref[...], 0.0)
        z = jnp.dot(y, w2_ref[...], preferred_element_type=jnp.float32) + b2_ref[...]
        o_ref[...] = z

    out = pl.pallas_call(
        body,
        grid=(_cdiv(n, block_rows),),
        in_specs=[
            pl.BlockSpec((block_rows, k), lambda i: (i, 0)),
            pl.BlockSpec((k, h), lambda i: (0, 0)),
            pl.BlockSpec((1, h), lambda i: (0, 0)),
            pl.BlockSpec((h, 1), lambda i: (0, 0)),
            pl.BlockSpec((1, 1), lambda i: (0, 0)),
        ],
        out_specs=pl.BlockSpec((block_rows, 1), lambda i: (i, 0)),
        out_shape=jax.ShapeDtypeStruct((n, 1), jnp.float32),
    )(x, w1, b1r, w2, b2r)
    return out[:, 0]


@functools.lru_cache(maxsize=None)
def _make_sc_msg_pass(hsize):
    hp = hsize // 128
    ppc = max(hp // 2, 1)
    e_core = N_EDGES if hp >= 2 else N_EDGES // 2
    e_tile = e_core // N_TILES
    n_chunks = e_tile // CHUNK
    out_slabs = hp if hp >= 2 else 2

    mesh = plsc.VectorSubcoreMesh(core_axis_name="c", subcore_axis_name="s")

    @functools.partial(
        pl.kernel,
        mesh=mesh,
        out_type=jax.ShapeDtypeStruct((out_slabs * N_NODES, 128), jnp.float32),
        scratch_types=[
            pltpu.VMEM((CHUNK,), jnp.int32),
            pltpu.VMEM((CHUNK,), jnp.int32),
            pltpu.VMEM((CHUNK,), jnp.int32),
            pltpu.VMEM((CHUNK, 128), jnp.float32),
            pltpu.VMEM((CHUNK, 128), jnp.float32),
            pltpu.VMEM((ZROWS, 128), jnp.float32),
            pltpu.VMEM_SHARED((N_NODES, 128), jnp.float32),
            pltpu.SemaphoreType.DMA,
        ],
    )
    def sc_kernel(hcur_hbm, gate_hbm, src_hbm, dst_hbm, out_hbm,
                  srcv, dstv, idxv, rows, gatev, zbuf, slab, sem):
        cid = lax.axis_index("c")
        sid = lax.axis_index("s")

        def zb_body(r, _):
            for k2 in range(8):
                zbuf[r, pl.ds(k2 * 16, 16)] = jnp.zeros((16,), jnp.float32)
            return 0
        lax.fori_loop(0, ZROWS, zb_body, 0)

        for pp in range(ppc):
            part = cid * ppc + pp

            for zi in range(ROWS_PER_TILE // ZROWS):
                pltpu.sync_copy(
                    zbuf, slab.at[pl.ds(sid * ROWS_PER_TILE + zi * ZROWS, ZROWS)])

            @pl.when(sid == N_TILES - 1)
            def _zero_tail():
                pltpu.sync_copy(zbuf.at[pl.ds(0, TAIL_ROWS)],
                                slab.at[pl.ds(ROWS_PER_TILE * N_TILES, TAIL_ROWS)])
            plsc.subcore_barrier()

            if hp >= 2:
                ebase = sid * e_tile
            else:
                ebase = cid * e_core + sid * e_tile

            def chunk_body(j, _):
                off = ebase + j * CHUNK
                pltpu.sync_copy(src_hbm.at[pl.ds(off, CHUNK)], srcv)
                pltpu.sync_copy(dst_hbm.at[pl.ds(off, CHUNK)], dstv)
                if hp >= 2:
                    def idx_body(k2, _):
                        sl = pl.ds(k2 * 16, 16)
                        idxv[sl] = srcv[sl] * hp + part
                        return 0
                    lax.fori_loop(0, CHUNK // 16, idx_body, 0)
                    gcp = pltpu.async_copy(hcur_hbm.at[idxv], rows, sem)
                    goff = part * N_EDGES + off
                else:
                    gcp = pltpu.async_copy(hcur_hbm.at[srcv], rows, sem)
                    goff = off
                pltpu.sync_copy(gate_hbm.at[pl.ds(goff, CHUNK)], gatev)
                gcp.wait()

                def mul_body(r, _):
                    for k2 in range(8):
                        sl = pl.ds(k2 * 16, 16)
                        rows[r, sl] = rows[r, sl] * gatev[r, sl]
                    return 0
                lax.fori_loop(0, CHUNK, mul_body, 0)

                pltpu.sync_copy(rows, slab.at[dstv], add=True)
                return 0
            lax.fori_loop(0, n_chunks, chunk_body, 0)
            plsc.subcore_barrier()

            slab_idx = part if hp >= 2 else cid
            pltpu.sync_copy(
                slab.at[pl.ds(sid * ROWS_PER_TILE, ROWS_PER_TILE)],
                out_hbm.at[pl.ds(slab_idx * N_NODES + sid * ROWS_PER_TILE,
                                 ROWS_PER_TILE)])

            @pl.when(sid == N_TILES - 1)
            def _dump_tail():
                pltpu.sync_copy(
                    slab.at[pl.ds(ROWS_PER_TILE * N_TILES, TAIL_ROWS)],
                    out_hbm.at[pl.ds(slab_idx * N_NODES + ROWS_PER_TILE * N_TILES,
                                     TAIL_ROWS)])

    return sc_kernel


def _egnn_block(p, src, dst, h, ef):
    w_in, b_in = p["in"]
    h0 = _fused_mm(h, w_in, b_in, act="relu")
    hcur = h0
    n, hsize = h0.shape
    hp = hsize // 128
    sc_call = _make_sc_msg_pass(hsize)
    for l, lp in enumerate(p["layers"]):
        we, be = lp["edge"]
        wl, bl = lp["lin"]
        gate = _gate_kernel(ef, we, be, hp)
        aggf = sc_call(hcur.reshape(n * hp, 128), gate, src, dst)
        hi = _hi_kernel(aggf, h0, hsize)
        beta = float(math.log(LAMDA / (l + 1) + 1.0))
        hcur = _lin_kernel(hi, wl, bl, beta)
    return hcur


def kernel(edge_index, h, x, adj, efeats, params):
    h = jnp.squeeze(h).astype(jnp.float32)
    x = jnp.squeeze(x)
    src, dst = edge_index[0], edge_index[1]
    dist = jnp.sum((x[src] - x[dst]) ** 2, axis=-1, keepdims=True)
    ef = jnp.concatenate([efeats, dist], axis=-1)

    f1 = _egnn_block(params["b1"], src, dst, h, ef)
    f2 = _egnn_block(params["b2"], src, dst, f1, ef)
    f3 = _egnn_block(params["b3"], src, dst, f2, ef)

    wm, bm = params["proj_middle"]
    middle_proj = _fused_mm(f2, wm, bm)
    ws, bs = params["proj_shallow"]
    shallow_proj = _fused_mm(f1, ws, bs)

    logit1 = _mlp2(f1, *params["cls1"][0], *params["cls1"][1])
    logit2 = _mlp2(f2, *params["cls2"][0], *params["cls2"][1])
    logit3 = _mlp2(f3, *params["cls3"][0], *params["cls3"][1])
    return ((logit3, logit2, logit1), (f3, middle_proj, shallow_proj))


# speedup vs baseline: 2.7381x; 1.4146x over previous
import functools
import math

import jax
import jax.numpy as jnp
from jax import lax
from jax.experimental import pallas as pl
from jax.experimental.pallas import tpu as pltpu
from jax.experimental.pallas import tpu_sc as plsc

ALPHA = 0.1
LAMDA = 0.5

N_NODES = 10000
N_EDGES = 320000
N_TILES = 16
ROWS_PER_TILE = 624
TAIL_ROWS = N_NODES - ROWS_PER_TILE * N_TILES
CHUNK = 80
ZROWS = 48


def _cdiv(a, b):
    return (a + b - 1) // b


def _fused_mm(x, w, b, act=None, block_rows=1024):
    n, k = x.shape
    m = w.shape[1]
    b2 = b.reshape(1, m)

    def body(x_ref, w_ref, b_ref, o_ref):
        y = jnp.dot(x_ref[...], w_ref[...], preferred_element_type=jnp.float32)
        y = y + b_ref[...]
        if act == "relu":
            y = jnp.maximum(y, 0.0)
        o_ref[...] = y

    return pl.pallas_call(
        body,
        grid=(_cdiv(n, block_rows),),
        in_specs=[
            pl.BlockSpec((block_rows, k), lambda i: (i, 0)),
            pl.BlockSpec((k, m), lambda i: (0, 0)),
            pl.BlockSpec((1, m), lambda i: (0, 0)),
        ],
        out_specs=pl.BlockSpec((block_rows, m), lambda i: (i, 0)),
        out_shape=jax.ShapeDtypeStruct((n, m), jnp.float32),
    )(x, w, b2)


def _gate_kernel(ef, we, be, hp, block_rows=8000):
    e, k = ef.shape
    nb = e // block_rows
    be2 = be.reshape(1, we.shape[1])

    def body(ef_ref, we_ref, be_ref, o_ref):
        y = jnp.dot(ef_ref[...], we_ref[...], preferred_element_type=jnp.float32)
        o_ref[...] = jax.nn.sigmoid(y + be_ref[...])

    return pl.pallas_call(
        body,
        grid=(hp, nb),
        in_specs=[
            pl.BlockSpec((block_rows, k), lambda p, i: (i, 0)),
            pl.BlockSpec((k, 128), lambda p, i: (0, p)),
            pl.BlockSpec((1, 128), lambda p, i: (0, p)),
        ],
        out_specs=pl.BlockSpec((block_rows, 128), lambda p, i: (p * nb + i, 0)),
        out_shape=jax.ShapeDtypeStruct((hp * e, 128), jnp.float32),
    )(ef, we, be2)


def _hi_kernel(aggf, h0, hsize, block_rows=1000):
    n, h = h0.shape
    nb = n // block_rows
    hp = hsize // 128

    if hp >= 2:
        def body(a_ref, h0_ref, o_ref):
            o_ref[...] = (1.0 - ALPHA) * a_ref[...] + ALPHA * h0_ref[...]

        return pl.pallas_call(
            body,
            grid=(nb, hp),
            in_specs=[
                pl.BlockSpec((block_rows, 128), lambda i, p: (p * nb + i, 0)),
                pl.BlockSpec((block_rows, 128), lambda i, p: (i, p)),
            ],
            out_specs=pl.BlockSpec((block_rows, 128), lambda i, p: (i, p)),
            out_shape=jax.ShapeDtypeStruct((n, h), jnp.float32),
        )(aggf, h0)

    def body2(a0_ref, a1_ref, h0_ref, o_ref):
        agg = a0_ref[...] + a1_ref[...]
        o_ref[...] = (1.0 - ALPHA) * agg + ALPHA * h0_ref[...]

    return pl.pallas_call(
        body2,
        grid=(nb,),
        in_specs=[
            pl.BlockSpec((block_rows, 128), lambda i: (i, 0)),
            pl.BlockSpec((block_rows, 128), lambda i: (nb + i, 0)),
            pl.BlockSpec((block_rows, 128), lambda i: (i, 0)),
        ],
        out_specs=pl.BlockSpec((block_rows, 128), lambda i: (i, 0)),
        out_shape=jax.ShapeDtypeStruct((n, h), jnp.float32),
    )(aggf, aggf, h0)


def _lin_kernel(hi, wl, bl, beta, block_rows=1000):
    n, h = hi.shape
    b2 = bl.reshape(1, h)

    def body(hi_ref, w_ref, b_ref, o_ref):
        hi_v = hi_ref[...]
        y = jnp.dot(hi_v, w_ref[...], preferred_element_type=jnp.float32) + b_ref[...]
        o_ref[...] = jnp.maximum((1.0 - beta) * hi_v + beta * y, 0.0)

    return pl.pallas_call(
        body,
        grid=(_cdiv(n, block_rows),),
        in_specs=[
            pl.BlockSpec((block_rows, h), lambda i: (i, 0)),
            pl.BlockSpec((h, h), lambda i: (0, 0)),
            pl.BlockSpec((1, h), lambda i: (0, 0)),
        ],
        out_specs=pl.BlockSpec((block_rows, h), lambda i: (i, 0)),
        out_shape=jax.ShapeDtypeStruct((n, h), jnp.float32),
    )(hi, wl, b2)


def _mlp2(x, w1, b1, w2, b2, block_rows=1000):
    n, k = x.shape
    h = w1.shape[1]
    b1r = b1.reshape(1, h)
    b2r = b2.reshape(1, 1)

    def body(x_ref, w1_ref, b1_ref, w2_ref, b2_ref, o_ref):
        y = jnp.dot(x_ref[...], w1_ref[...], preferred_element_type=jnp.float32)
        y = jnp.maximum(y + b1_ref[...], 0.0)
        z = jnp.dot(y, w2_ref[...], preferred_element_type=jnp.float32) + b2_ref[...]
        o_ref[...] = z

    out = pl.pallas_call(
        body,
        grid=(_cdiv(n, block_rows),),
        in_specs=[
            pl.BlockSpec((block_rows, k), lambda i: (i, 0)),
            pl.BlockSpec((k, h), lambda i: (0, 0)),
            pl.BlockSpec((1, h), lambda i: (0, 0)),
            pl.BlockSpec((h, 1), lambda i: (0, 0)),
            pl.BlockSpec((1, 1), lambda i: (0, 0)),
        ],
        out_specs=pl.BlockSpec((block_rows, 1), lambda i: (i, 0)),
        out_shape=jax.ShapeDtypeStruct((n, 1), jnp.float32),
    )(x, w1, b1r, w2, b2r)
    return out[:, 0]


@functools.lru_cache(maxsize=None)
def _make_sc_msg_pass(hsize):
    hp = hsize // 128
    ppc = max(hp // 2, 1)
    e_core = N_EDGES if hp >= 2 else N_EDGES // 2
    e_tile = e_core // N_TILES
    n_chunks = e_tile // CHUNK
    out_slabs = hp if hp >= 2 else 2

    mesh = plsc.VectorSubcoreMesh(core_axis_name="c", subcore_axis_name="s")

    @functools.partial(
        pl.kernel,
        mesh=mesh,
        out_type=jax.ShapeDtypeStruct((out_slabs * N_NODES, 128), jnp.float32),
        scratch_types=[
            pltpu.VMEM((CHUNK,), jnp.int32),
            pltpu.VMEM((CHUNK,), jnp.int32),
            pltpu.VMEM((CHUNK,), jnp.int32),
            pltpu.VMEM((CHUNK,), jnp.int32),
            pltpu.VMEM((CHUNK, 128), jnp.float32),
            pltpu.VMEM((CHUNK, 128), jnp.float32),
            pltpu.VMEM((CHUNK,), jnp.int32),
            pltpu.VMEM((CHUNK,), jnp.int32),
            pltpu.VMEM((CHUNK,), jnp.int32),
            pltpu.VMEM((CHUNK,), jnp.int32),
            pltpu.VMEM((CHUNK, 128), jnp.float32),
            pltpu.VMEM((CHUNK, 128), jnp.float32),
            pltpu.VMEM((ZROWS, 128), jnp.float32),
            pltpu.VMEM_SHARED((N_NODES, 128), jnp.float32),
            pltpu.SemaphoreType.DMA,
            pltpu.SemaphoreType.DMA,
            pltpu.SemaphoreType.DMA,
            pltpu.SemaphoreType.DMA,
        ],
    )
    def sc_kernel(hcur_hbm, gate_hbm, src_hbm, dst_hbm, out_hbm,
                  srcA, dstA, idxA, sctA, rowsA, gateA,
                  srcB, dstB, idxB, sctB, rowsB, gateB,
                  zbuf, slab, semA, semB, gsemA, gsemB):
        cid = lax.axis_index("c")
        sid = lax.axis_index("s")

        def zb_body(r, _):
            for k2 in range(8):
                zbuf[r, pl.ds(k2 * 16, 16)] = jnp.zeros((16,), jnp.float32)
            return 0
        lax.fori_loop(0, ZROWS, zb_body, 0)

        bufs = (
            (srcA, dstA, idxA, sctA, rowsA, gateA, semA, gsemA),
            (srcB, dstB, idxB, sctB, rowsB, gateB, semB, gsemB),
        )

        for pp in range(ppc):
            part = cid * ppc + pp if hp >= 2 else 0

            for zi in range(ROWS_PER_TILE // ZROWS):
                pltpu.sync_copy(
                    zbuf, slab.at[pl.ds(sid * ROWS_PER_TILE + zi * ZROWS, ZROWS)])

            @pl.when(sid == N_TILES - 1)
            def _zero_tail():
                pltpu.sync_copy(zbuf.at[pl.ds(0, TAIL_ROWS)],
                                slab.at[pl.ds(ROWS_PER_TILE * N_TILES, TAIL_ROWS)])
            plsc.subcore_barrier()

            if hp >= 2:
                ebase = sid * e_tile
            else:
                ebase = cid * e_core + sid * e_tile

            def issue_in(c, bi):
                srcv, dstv, _, _, _, gatev, sem, _ = bufs[bi]
                off = ebase + c * CHUNK
                pltpu.async_copy(src_hbm.at[pl.ds(off, CHUNK)], srcv, sem)
                pltpu.async_copy(dst_hbm.at[pl.ds(off, CHUNK)], dstv, sem)
                goff = part * N_EDGES + off
                pltpu.async_copy(gate_hbm.at[pl.ds(goff, CHUNK)], gatev, sem)

            def start_gather(bi):
                srcv, dstv, idxv, _, rowsv, gatev, sem, gsem = bufs[bi]
                pltpu.make_async_copy(src_hbm.at[pl.ds(0, CHUNK)], srcv, sem).wait()
                pltpu.make_async_copy(dst_hbm.at[pl.ds(0, CHUNK)], dstv, sem).wait()
                pltpu.make_async_copy(gate_hbm.at[pl.ds(0, CHUNK)], gatev, sem).wait()

                def idx_body(k2, _):
                    sl = pl.ds(k2 * 16, 16)
                    idxv[sl] = srcv[sl] * hp + part
                    return 0
                lax.fori_loop(0, CHUNK // 16, idx_body, 0)
                pltpu.async_copy(hcur_hbm.at[idxv], rowsv, gsem)

            def finish_chunk(c, bi):
                _, dstv, idxv, sctv, rowsv, gatev, _, gsem = bufs[bi]
                pltpu.make_async_copy(hcur_hbm.at[idxv], rowsv, gsem).wait()

                def mul_body(r, _):
                    for rr in range(4):
                        for k2 in range(8):
                            sl = pl.ds(k2 * 16, 16)
                            rowsv[4 * r + rr, sl] = (rowsv[4 * r + rr, sl]
                                                     * gatev[4 * r + rr, sl])
                    return 0
                lax.fori_loop(0, CHUNK // 4, mul_body, 0)

                def sct_body(k2, _):
                    sl = pl.ds(k2 * 16, 16)
                    sctv[sl] = dstv[sl]
                    return 0
                lax.fori_loop(0, CHUNK // 16, sct_body, 0)

                @pl.when(c + 2 < n_chunks)
                def _prefetch():
                    issue_in(c + 2, bi)
                pltpu.sync_copy(rowsv, slab.at[sctv], add=True)

            def half(c, finish_bi):
                @pl.when(c + 1 < n_chunks)
                def _fire():
                    start_gather(1 - finish_bi)

                @pl.when(c < n_chunks)
                def _fin():
                    finish_chunk(c, finish_bi)

            issue_in(0, 0)
            start_gather(0)
            issue_in(1, 1)

            def pair_body(j, _):
                half(2 * j, 0)
                half(2 * j + 1, 1)
                return 0
            lax.fori_loop(0, (n_chunks + 1) // 2, pair_body, 0)
            plsc.subcore_barrier()

            slab_idx = part if hp >= 2 else cid
            pltpu.sync_copy(
                slab.at[pl.ds(sid * ROWS_PER_TILE, ROWS_PER_TILE)],
                out_hbm.at[pl.ds(slab_idx * N_NODES + sid * ROWS_PER_TILE,
                                 ROWS_PER_TILE)])

            @pl.when(sid == N_TILES - 1)
            def _dump_tail():
                pltpu.sync_copy(
                    slab.at[pl.ds(ROWS_PER_TILE * N_TILES, TAIL_ROWS)],
                    out_hbm.at[pl.ds(slab_idx * N_NODES + ROWS_PER_TILE * N_TILES,
                                     TAIL_ROWS)])

    return sc_kernel


def _egnn_block(p, src, dst, h, ef):
    w_in, b_in = p["in"]
    h0 = _fused_mm(h, w_in, b_in, act="relu")
    hcur = h0
    n, hsize = h0.shape
    hp = hsize // 128
    sc_call = _make_sc_msg_pass(hsize)
    for l, lp in enumerate(p["layers"]):
        we, be = lp["edge"]
        wl, bl = lp["lin"]
        gate = _gate_kernel(ef, we, be, hp)
        aggf = sc_call(hcur.reshape(n * hp, 128), gate, src, dst)
        hi = _hi_kernel(aggf, h0, hsize)
        beta = float(math.log(LAMDA / (l + 1) + 1.0))
        hcur = _lin_kernel(hi, wl, bl, beta)
    return hcur


def kernel(edge_index, h, x, adj, efeats, params):
    h = jnp.squeeze(h).astype(jnp.float32)
    x = jnp.squeeze(x)
    src, dst = edge_index[0], edge_index[1]
    dist = jnp.sum((x[src] - x[dst]) ** 2, axis=-1, keepdims=True)
    ef = jnp.concatenate([efeats, dist], axis=-1)

    f1 = _egnn_block(params["b1"], src, dst, h, ef)
    f2 = _egnn_block(params["b2"], src, dst, f1, ef)
    f3 = _egnn_block(params["b3"], src, dst, f2, ef)

    wm, bm = params["proj_middle"]
    middle_proj = _fused_mm(f2, wm, bm)
    ws, bs = params["proj_shallow"]
    shallow_proj = _fused_mm(f1, ws, bs)

    logit1 = _mlp2(f1, *params["cls1"][0], *params["cls1"][1])
    logit2 = _mlp2(f2, *params["cls2"][0], *params["cls2"][1])
    logit3 = _mlp2(f3, *params["cls3"][0], *params["cls3"][1])
    return ((logit3, logit2, logit1), (f3, middle_proj, shallow_proj))
